# baseline (speedup 1.0000x reference)
import functools

import jax
import jax.numpy as jnp
from jax.experimental import pallas as pl
from jax.experimental.pallas import tpu as pltpu

_VMEM_LIMIT = 40 * 1024 * 1024


def _block_kernel(x_ref, w1_ref, b1_ref, w2_ref, b2_ref, w3_ref, b3_ref,
                  *rest, nb, H, W, Ci, Cm, Co, stride, has_ds):
    if has_ds:
        wd_ref, bd_ref, o_ref = rest
    else:
        (o_ref,) = rest
    OH, OW = H // stride, W // stride

    x = x_ref[...]
    xm = x.reshape(nb * H * W, Ci)

    h1 = jnp.dot(xm, w1_ref[...], preferred_element_type=jnp.float32)
    h1 = jnp.maximum(h1 + b1_ref[...], 0.0).astype(jnp.bfloat16)
    h1 = h1.reshape(nb, H, W, Cm)

    h1p = jnp.pad(h1, ((0, 0), (1, 1), (1, 1), (0, 0)))
    if stride == 1:
        parts = [h1p[:, ki:ki + OH, kj:kj + OW, :]
                 for ki in range(3) for kj in range(3)]
    else:
        Hh, Wh = (H + 2) // 2, (W + 2) // 2
        h1r = h1p.reshape(nb, Hh, 2, Wh, 2, Cm)
        P = [[h1r[:, :, rp, :, cp, :] for cp in (0, 1)] for rp in (0, 1)]
        sel = ((0, 0), (1, 0), (0, 1))
        parts = []
        for ki in range(3):
            rp, ro = sel[ki]
            for kj in range(3):
                cp, co = sel[kj]
                parts.append(P[rp][cp][:, ro:ro + OH, co:co + OW, :])
    a = jnp.concatenate(parts, axis=-1).reshape(nb * OH * OW, 9 * Cm)
    h2 = jnp.dot(a, w2_ref[...], preferred_element_type=jnp.float32)
    h2 = jnp.maximum(h2 + b2_ref[...], 0.0).astype(jnp.bfloat16)

    if has_ds:
        if stride == 2:
            xr = x.reshape(nb, OH, 2, OW, 2, Ci)
            xs = xr[:, :, 0, :, 0, :].reshape(nb * OH * OW, Ci)
        else:
            xs = xm
        ident = jnp.dot(xs, wd_ref[...], preferred_element_type=jnp.float32)
        ident = (ident + bd_ref[...]).astype(jnp.bfloat16)
    else:
        ident = xm

    y = jnp.dot(h2, w3_ref[...], preferred_element_type=jnp.float32)
    y = y + b3_ref[...] + ident.astype(jnp.float32)
    o_ref[...] = jnp.maximum(y, 0.0).astype(jnp.bfloat16).reshape(
        nb, OH, OW, Co)


def _bottleneck(x, w1, b1, w2, b2, w3, b3, wd=None, bd=None, *,
                stride=1, nb=1):
    B, H, W, Ci = x.shape
    nb = min(nb, B)
    Cm = w1.shape[-1]
    Co = w3.shape[-1]
    OH, OW = H // stride, W // stride
    has_ds = wd is not None

    operands = [
        x,
        w1.reshape(Ci, Cm).astype(jnp.bfloat16),
        b1.astype(jnp.float32).reshape(1, Cm),
        w2.reshape(9 * Cm, Cm).astype(jnp.bfloat16),
        b2.astype(jnp.float32).reshape(1, Cm),
        w3.reshape(Cm, Co).astype(jnp.bfloat16),
        b3.astype(jnp.float32).reshape(1, Co),
    ]
    in_specs = [
        pl.BlockSpec((nb, H, W, Ci), lambda b: (b, 0, 0, 0)),
        pl.BlockSpec((Ci, Cm), lambda b: (0, 0)),
        pl.BlockSpec((1, Cm), lambda b: (0, 0)),
        pl.BlockSpec((9 * Cm, Cm), lambda b: (0, 0)),
        pl.BlockSpec((1, Cm), lambda b: (0, 0)),
        pl.BlockSpec((Cm, Co), lambda b: (0, 0)),
        pl.BlockSpec((1, Co), lambda b: (0, 0)),
    ]
    if has_ds:
        operands.append(wd.reshape(Ci, Co).astype(jnp.bfloat16))
        operands.append(bd.astype(jnp.float32).reshape(1, Co))
        in_specs.append(pl.BlockSpec((Ci, Co), lambda b: (0, 0)))
        in_specs.append(pl.BlockSpec((1, Co), lambda b: (0, 0)))

    return pl.pallas_call(
        functools.partial(_block_kernel, nb=nb, H=H, W=W, Ci=Ci, Cm=Cm,
                          Co=Co, stride=stride, has_ds=has_ds),
        out_shape=jax.ShapeDtypeStruct((B, OH, OW, Co), jnp.bfloat16),
        grid=(B // nb,),
        in_specs=in_specs,
        out_specs=pl.BlockSpec((nb, OH, OW, Co), lambda b: (b, 0, 0, 0)),
        compiler_params=pltpu.CompilerParams(
            dimension_semantics=("parallel",),
            vmem_limit_bytes=_VMEM_LIMIT),
    )(*operands)


def _stem_kernel(xs_ref, w_ref, b_ref, o_ref):
    t = pl.program_id(1)
    xst = xs_ref[pl.ds(28 * t, 32)]
    parts = [xst[a:a + 29, b:b + 112, :]
             for a in range(4) for b in range(4)]
    av = jnp.concatenate(parts, axis=-1).reshape(29 * 112, 192)
    y = jnp.dot(av, w_ref[...], preferred_element_type=jnp.float32)
    y = jnp.maximum(y + b_ref[...], 0.0).astype(jnp.bfloat16)
    y = y.reshape(29, 112, 128)
    m = jnp.where(t == 0, 0.0, 1.0).astype(jnp.bfloat16)
    y = jnp.concatenate([y[0:1] * m, y[1:]], axis=0)

    ya = y[0:28].reshape(14, 2, 112, 128)
    yc = y[1:29].reshape(14, 2, 112, 128)
    rm = jnp.maximum(jnp.maximum(ya[:, 0], ya[:, 1]), yc[:, 1])
    rr = rm.reshape(14, 56, 2, 128)
    ce, co = rr[:, :, 0, :], rr[:, :, 1, :]
    zcol = jnp.zeros((14, 1, 128), jnp.bfloat16)
    co_l = jnp.concatenate([zcol, co[:, :-1]], axis=1)
    o_ref[...] = jnp.maximum(jnp.maximum(ce, co), co_l)


def _stem_pool(x_nhwc, stem_w, stem_b):
    B = x_nhwc.shape[0]
    xp = jnp.pad(x_nhwc, ((0, 0), (3, 3), (3, 3), (0, 0)))
    xs = xp.reshape(B, 115, 2, 115, 2, 3).transpose(0, 1, 3, 2, 4, 5)
    xs = xs.reshape(B, 115, 115, 12)
    xs = jnp.pad(xs, ((0, 0), (1, 0), (0, 0), (0, 0)))
    wp = jnp.pad(stem_w.astype(jnp.bfloat16),
                 ((0, 1), (0, 1), (0, 0), (0, 0)))
    ws = wp.reshape(4, 2, 4, 2, 3, 128).transpose(0, 2, 1, 3, 4, 5)
    ws = ws.reshape(192, 128)
    bs = stem_b.astype(jnp.float32).reshape(1, 128)

    return pl.pallas_call(
        _stem_kernel,
        out_shape=jax.ShapeDtypeStruct((B, 56, 56, 128), jnp.bfloat16),
        grid=(B, 4),
        in_specs=[
            pl.BlockSpec((None, 116, 115, 12), lambda b, t: (b, 0, 0, 0)),
            pl.BlockSpec((192, 128), lambda b, t: (0, 0)),
            pl.BlockSpec((1, 128), lambda b, t: (0, 0)),
        ],
        out_specs=pl.BlockSpec((None, 14, 56, 128), lambda b, t: (b, t, 0, 0)),
        compiler_params=pltpu.CompilerParams(
            dimension_semantics=("parallel", "arbitrary"),
            vmem_limit_bytes=_VMEM_LIMIT),
    )(xs, ws, bs)


def _gap_fc_kernel(x_ref, w_ref, b_ref, o_ref, *, inv_hw):
    pooled = jnp.sum(x_ref[...].astype(jnp.float32), axis=1) * inv_hw
    o_ref[...] = jnp.dot(pooled.astype(jnp.bfloat16), w_ref[...],
                         preferred_element_type=jnp.float32) + b_ref[...]


def _gap_fc(x_nhwc, fc_w, fc_b, num_classes, nb=8):
    B, H, W, C = x_nhwc.shape
    nb = min(nb, B)
    x3 = x_nhwc.reshape(B, H * W, C)
    Np = 256
    w_p = jnp.pad(fc_w.astype(jnp.bfloat16), ((0, 0), (0, Np - num_classes)))
    b_p = jnp.pad(fc_b.astype(jnp.float32), (0, Np - num_classes))
    b_p = b_p.reshape(1, Np)
    out = pl.pallas_call(
        functools.partial(_gap_fc_kernel, inv_hw=1.0 / float(H * W)),
        out_shape=jax.ShapeDtypeStruct((B, Np), jnp.float32),
        grid=(B // nb,),
        in_specs=[
            pl.BlockSpec((nb, H * W, C), lambda b: (b, 0, 0)),
            pl.BlockSpec((C, Np), lambda b: (0, 0)),
            pl.BlockSpec((1, Np), lambda b: (0, 0)),
        ],
        out_specs=pl.BlockSpec((nb, Np), lambda b: (b, 0)),
        compiler_params=pltpu.CompilerParams(
            dimension_semantics=("parallel",),
            vmem_limit_bytes=_VMEM_LIMIT),
    )(x3, w_p, b_p)
    return out[:, :num_classes]


def kernel(x, stem_w, stem_b, s0_b0_c1_w, s0_b0_c1_b, s0_b0_c2_w, s0_b0_c2_b, s0_b0_c3_w, s0_b0_c3_b, s0_b0_ds_w, s0_b0_ds_b, s0_b1_c1_w, s0_b1_c1_b, s0_b1_c2_w, s0_b1_c2_b, s0_b1_c3_w, s0_b1_c3_b, s0_b2_c1_w, s0_b2_c1_b, s0_b2_c2_w, s0_b2_c2_b, s0_b2_c3_w, s0_b2_c3_b, s1_b0_c1_w, s1_b0_c1_b, s1_b0_c2_w, s1_b0_c2_b, s1_b0_c3_w, s1_b0_c3_b, s1_b0_ds_w, s1_b0_ds_b, s1_b1_c1_w, s1_b1_c1_b, s1_b1_c2_w, s1_b1_c2_b, s1_b1_c3_w, s1_b1_c3_b, s1_b2_c1_w, s1_b2_c1_b, s1_b2_c2_w, s1_b2_c2_b, s1_b2_c3_w, s1_b2_c3_b, s1_b3_c1_w, s1_b3_c1_b, s1_b3_c2_w, s1_b3_c2_b, s1_b3_c3_w, s1_b3_c3_b, s2_b0_c1_w, s2_b0_c1_b, s2_b0_c2_w, s2_b0_c2_b, s2_b0_c3_w, s2_b0_c3_b, s2_b0_ds_w, s2_b0_ds_b, s2_b1_c1_w, s2_b1_c1_b, s2_b1_c2_w, s2_b1_c2_b, s2_b1_c3_w, s2_b1_c3_b, s2_b2_c1_w, s2_b2_c1_b, s2_b2_c2_w, s2_b2_c2_b, s2_b2_c3_w, s2_b2_c3_b, s2_b3_c1_w, s2_b3_c1_b, s2_b3_c2_w, s2_b3_c2_b, s2_b3_c3_w, s2_b3_c3_b, s2_b4_c1_w, s2_b4_c1_b, s2_b4_c2_w, s2_b4_c2_b, s2_b4_c3_w, s2_b4_c3_b, s2_b5_c1_w, s2_b5_c1_b, s2_b5_c2_w, s2_b5_c2_b, s2_b5_c3_w, s2_b5_c3_b, s3_b0_c1_w, s3_b0_c1_b, s3_b0_c2_w, s3_b0_c2_b, s3_b0_c3_w, s3_b0_c3_b, s3_b0_ds_w, s3_b0_ds_b, s3_b1_c1_w, s3_b1_c1_b, s3_b1_c2_w, s3_b1_c2_b, s3_b1_c3_w, s3_b1_c3_b, s3_b2_c1_w, s3_b2_c1_b, s3_b2_c2_w, s3_b2_c2_b, s3_b2_c3_w, s3_b2_c3_b, fc_w, fc_b):
    A = dict(locals())
    t = jnp.transpose(x, (0, 2, 3, 1)).astype(jnp.bfloat16)
    t = _stem_pool(t, stem_w, stem_b)

    n_blocks = (3, 4, 6, 3)
    strides = (1, 2, 2, 2)
    batch_group = ((1, 1), (1, 2), (2, 4), (4, 8))
    for si in range(4):
        for bi in range(n_blocks[si]):
            args = [A[f's{si}_b{bi}_{c}_{t2}'] for c in ('c1', 'c2', 'c3')
                    for t2 in ('w', 'b')]
            if bi == 0:
                t = _bottleneck(t, *args, A[f's{si}_b{bi}_ds_w'],
                                A[f's{si}_b{bi}_ds_b'],
                                stride=strides[si], nb=batch_group[si][0])
            else:
                t = _bottleneck(t, *args, stride=1, nb=batch_group[si][1])

    return _gap_fc(t, fc_w, fc_b, 200)


# speedup vs baseline: 1.8870x; 1.8870x over previous
import functools

import jax
import jax.numpy as jnp
from jax.experimental import pallas as pl
from jax.experimental.pallas import tpu as pltpu

_VMEM_LIMIT = 40 * 1024 * 1024


def _block_kernel(x_ref, w1_ref, b1_ref, w2_ref, b2_ref, w3_ref, b3_ref,
                  *rest, nb, H, W, Ci, Cm, Co, stride, has_ds, flat_in,
                  flat_out):
    if has_ds:
        wd_ref, bd_ref, o_ref = rest
    else:
        (o_ref,) = rest
    OH, OW = H // stride, W // stride

    if flat_in:
        xm = x_ref[...]
        x = None
    else:
        x = x_ref[...]
        xm = x.reshape(nb * H * W, Ci)

    h1 = jnp.dot(xm, w1_ref[...], preferred_element_type=jnp.float32)
    h1 = jnp.maximum(h1 + b1_ref[...], 0.0).astype(jnp.bfloat16)
    h1 = h1.reshape(nb, H, W, Cm)

    h1p = jnp.pad(h1, ((0, 0), (1, 1), (1, 1), (0, 0)))
    if stride == 1 and W % 8 == 0:
        akj = jnp.concatenate([h1p[:, :, kj:kj + OW, :] for kj in range(3)],
                              axis=-1)
        h2 = None
        for ki in range(3):
            p = akj[:, ki:ki + OH].reshape(nb * OH * OW, 3 * Cm)
            d = jnp.dot(p, w2_ref[ki * 3 * Cm:(ki + 1) * 3 * Cm, :],
                        preferred_element_type=jnp.float32)
            h2 = d if h2 is None else h2 + d
        h2 = jnp.maximum(h2 + b2_ref[...], 0.0).astype(jnp.bfloat16)
    elif stride == 1:
        parts = [h1p[:, ki:ki + OH, kj:kj + OW, :]
                 for ki in range(3) for kj in range(3)]
    else:
        Hh, Wh = (H + 2) // 2, (W + 2) // 2
        h1r = h1p.reshape(nb, Hh, 2, Wh, 2, Cm)
        P = [[h1r[:, :, rp, :, cp, :] for cp in (0, 1)] for rp in (0, 1)]
        sel = ((0, 0), (1, 0), (0, 1))
        parts = []
        for ki in range(3):
            rp, ro = sel[ki]
            for kj in range(3):
                cp, co = sel[kj]
                parts.append(P[rp][cp][:, ro:ro + OH, co:co + OW, :])
    if not (stride == 1 and W % 8 == 0):
        a = jnp.concatenate(parts, axis=-1).reshape(nb * OH * OW, 9 * Cm)
        h2 = jnp.dot(a, w2_ref[...], preferred_element_type=jnp.float32)
        h2 = jnp.maximum(h2 + b2_ref[...], 0.0).astype(jnp.bfloat16)

    if has_ds:
        if stride == 2:
            x4 = xm.reshape(nb, H, W, Ci) if flat_in else x
            xr = x4.reshape(nb, OH, 2, OW, 2, Ci)
            xs = xr[:, :, 0, :, 0, :].reshape(nb * OH * OW, Ci)
        else:
            xs = xm
        ident = jnp.dot(xs, wd_ref[...], preferred_element_type=jnp.float32)
        ident = (ident + bd_ref[...]).astype(jnp.bfloat16)
    else:
        ident = xm

    y = jnp.dot(h2, w3_ref[...], preferred_element_type=jnp.float32)
    y = y + b3_ref[...] + ident.astype(jnp.float32)
    y = jnp.maximum(y, 0.0).astype(jnp.bfloat16)
    o_ref[...] = y if flat_out else y.reshape(nb, OH, OW, Co)


def _bottleneck(x, w1, b1, w2, b2, w3, b3, wd=None, bd=None, *,
                stride=1, nb=1, hw=None, flat_out=False):
    flat_in = hw is not None
    if flat_in:
        H, W = hw
        B = x.shape[0] // (H * W)
        Ci = x.shape[-1]
    else:
        B, H, W, Ci = x.shape
    nb = min(nb, B)
    Cm = w1.shape[-1]
    Co = w3.shape[-1]
    OH, OW = H // stride, W // stride
    has_ds = wd is not None

    operands = [
        x,
        w1.reshape(Ci, Cm).astype(jnp.bfloat16),
        b1.astype(jnp.float32).reshape(1, Cm),
        w2.reshape(9 * Cm, Cm).astype(jnp.bfloat16),
        b2.astype(jnp.float32).reshape(1, Cm),
        w3.reshape(Cm, Co).astype(jnp.bfloat16),
        b3.astype(jnp.float32).reshape(1, Co),
    ]
    in_specs = [
        pl.BlockSpec((nb * H * W, Ci), lambda b: (b, 0)) if flat_in
        else pl.BlockSpec((nb, H, W, Ci), lambda b: (b, 0, 0, 0)),
        pl.BlockSpec((Ci, Cm), lambda b: (0, 0)),
        pl.BlockSpec((1, Cm), lambda b: (0, 0)),
        pl.BlockSpec((9 * Cm, Cm), lambda b: (0, 0)),
        pl.BlockSpec((1, Cm), lambda b: (0, 0)),
        pl.BlockSpec((Cm, Co), lambda b: (0, 0)),
        pl.BlockSpec((1, Co), lambda b: (0, 0)),
    ]
    if has_ds:
        operands.append(wd.reshape(Ci, Co).astype(jnp.bfloat16))
        operands.append(bd.astype(jnp.float32).reshape(1, Co))
        in_specs.append(pl.BlockSpec((Ci, Co), lambda b: (0, 0)))
        in_specs.append(pl.BlockSpec((1, Co), lambda b: (0, 0)))

    if flat_out:
        out_shape = jax.ShapeDtypeStruct((B * OH * OW, Co), jnp.bfloat16)
        out_spec = pl.BlockSpec((nb * OH * OW, Co), lambda b: (b, 0))
    else:
        out_shape = jax.ShapeDtypeStruct((B, OH, OW, Co), jnp.bfloat16)
        out_spec = pl.BlockSpec((nb, OH, OW, Co), lambda b: (b, 0, 0, 0))
    return pl.pallas_call(
        functools.partial(_block_kernel, nb=nb, H=H, W=W, Ci=Ci, Cm=Cm,
                          Co=Co, stride=stride, has_ds=has_ds,
                          flat_in=flat_in, flat_out=flat_out),
        out_shape=out_shape,
        grid=(B // nb,),
        in_specs=in_specs,
        out_specs=out_spec,
        compiler_params=pltpu.CompilerParams(
            dimension_semantics=("parallel",),
            vmem_limit_bytes=_VMEM_LIMIT),
    )(*operands)


def _stem_kernel(v00_ref, v02_ref, v20_ref, v22_ref, w_ref, b_ref, o_ref):
    def phase(v_ref):
        v = v_ref[...]
        ab = jnp.concatenate([v[:, b2:b2 + 56, :] for b2 in range(2)],
                             axis=-1)
        y = None
        for a in range(2):
            p = ab[a:a + 56].reshape(3136, 96)
            d = jnp.dot(p, w_ref[a * 96:(a + 1) * 96, :],
                        preferred_element_type=jnp.float32)
            y = d if y is None else y + d
        y = jnp.maximum(y + b_ref[...], 0.0).astype(jnp.bfloat16)
        return y.reshape(56, 56, 128)

    ee = phase(v00_ref)
    eo = phase(v02_ref)
    oe = phase(v20_ref)
    oo = phase(v22_ref)
    zc = jnp.zeros((56, 1, 128), jnp.bfloat16)
    eo_l = jnp.concatenate([zc, eo[:, :-1]], axis=1)
    oo_l = jnp.concatenate([zc, oo[:, :-1]], axis=1)
    ce = jnp.maximum(jnp.maximum(ee, eo), eo_l)
    co = jnp.maximum(jnp.maximum(oe, oo), oo_l)
    zr = jnp.zeros((1, 56, 128), jnp.bfloat16)
    co_u = jnp.concatenate([zr, co[:-1]], axis=0)
    o_ref[...] = jnp.maximum(jnp.maximum(ce, co), co_u)


def _stem_pool(x_nhwc, stem_w, stem_b):
    B = x_nhwc.shape[0]
    xp = jnp.pad(x_nhwc, ((0, 0), (3, 7), (3, 7), (0, 0)))

    def s4d(dr, dc):
        q = xp[:, dr:dr + 232, dc:dc + 232, :]
        q = q.reshape(B, 58, 4, 58, 4, 3).transpose(0, 1, 3, 2, 4, 5)
        return q.reshape(B, 58, 58, 48)

    views = [s4d(0, 0), s4d(0, 2), s4d(2, 0), s4d(2, 2)]
    wp = jnp.pad(stem_w.astype(jnp.bfloat16),
                 ((0, 1), (0, 1), (0, 0), (0, 0)))
    ws = wp.reshape(2, 4, 2, 4, 3, 128).transpose(0, 2, 1, 3, 4, 5)
    ws = ws.reshape(192, 128)
    bs = stem_b.astype(jnp.float32).reshape(1, 128)

    vspec = pl.BlockSpec((None, 58, 58, 48), lambda b: (b, 0, 0, 0))
    return pl.pallas_call(
        _stem_kernel,
        out_shape=jax.ShapeDtypeStruct((B, 56, 56, 128), jnp.bfloat16),
        grid=(B,),
        in_specs=[vspec, vspec, vspec, vspec,
                  pl.BlockSpec((192, 128), lambda b: (0, 0)),
                  pl.BlockSpec((1, 128), lambda b: (0, 0))],
        out_specs=pl.BlockSpec((None, 56, 56, 128), lambda b: (b, 0, 0, 0)),
        compiler_params=pltpu.CompilerParams(
            dimension_semantics=("parallel",),
            vmem_limit_bytes=_VMEM_LIMIT),
    )(*views, ws, bs)


def _gap_fc_kernel(x_ref, w_ref, b_ref, o_ref, *, inv_hw):
    pooled = jnp.sum(x_ref[...].astype(jnp.float32), axis=1) * inv_hw
    o_ref[...] = jnp.dot(pooled.astype(jnp.bfloat16), w_ref[...],
                         preferred_element_type=jnp.float32) + b_ref[...]


def _gap_fc(x_nhwc, fc_w, fc_b, num_classes, nb=8):
    if x_nhwc.ndim == 3:
        B, HW, C = x_nhwc.shape
        H, W = HW, 1
        x3 = x_nhwc
    else:
        B, H, W, C = x_nhwc.shape
        x3 = x_nhwc.reshape(B, H * W, C)
    nb = min(nb, B)
    Np = 256
    w_p = jnp.pad(fc_w.astype(jnp.bfloat16), ((0, 0), (0, Np - num_classes)))
    b_p = jnp.pad(fc_b.astype(jnp.float32), (0, Np - num_classes))
    b_p = b_p.reshape(1, Np)
    out = pl.pallas_call(
        functools.partial(_gap_fc_kernel, inv_hw=1.0 / float(H * W)),
        out_shape=jax.ShapeDtypeStruct((B, Np), jnp.float32),
        grid=(B // nb,),
        in_specs=[
            pl.BlockSpec((nb, H * W, C), lambda b: (b, 0, 0)),
            pl.BlockSpec((C, Np), lambda b: (0, 0)),
            pl.BlockSpec((1, Np), lambda b: (0, 0)),
        ],
        out_specs=pl.BlockSpec((nb, Np), lambda b: (b, 0)),
        compiler_params=pltpu.CompilerParams(
            dimension_semantics=("parallel",),
            vmem_limit_bytes=_VMEM_LIMIT),
    )(x3, w_p, b_p)
    return out[:, :num_classes]


def kernel(x, stem_w, stem_b, s0_b0_c1_w, s0_b0_c1_b, s0_b0_c2_w, s0_b0_c2_b, s0_b0_c3_w, s0_b0_c3_b, s0_b0_ds_w, s0_b0_ds_b, s0_b1_c1_w, s0_b1_c1_b, s0_b1_c2_w, s0_b1_c2_b, s0_b1_c3_w, s0_b1_c3_b, s0_b2_c1_w, s0_b2_c1_b, s0_b2_c2_w, s0_b2_c2_b, s0_b2_c3_w, s0_b2_c3_b, s1_b0_c1_w, s1_b0_c1_b, s1_b0_c2_w, s1_b0_c2_b, s1_b0_c3_w, s1_b0_c3_b, s1_b0_ds_w, s1_b0_ds_b, s1_b1_c1_w, s1_b1_c1_b, s1_b1_c2_w, s1_b1_c2_b, s1_b1_c3_w, s1_b1_c3_b, s1_b2_c1_w, s1_b2_c1_b, s1_b2_c2_w, s1_b2_c2_b, s1_b2_c3_w, s1_b2_c3_b, s1_b3_c1_w, s1_b3_c1_b, s1_b3_c2_w, s1_b3_c2_b, s1_b3_c3_w, s1_b3_c3_b, s2_b0_c1_w, s2_b0_c1_b, s2_b0_c2_w, s2_b0_c2_b, s2_b0_c3_w, s2_b0_c3_b, s2_b0_ds_w, s2_b0_ds_b, s2_b1_c1_w, s2_b1_c1_b, s2_b1_c2_w, s2_b1_c2_b, s2_b1_c3_w, s2_b1_c3_b, s2_b2_c1_w, s2_b2_c1_b, s2_b2_c2_w, s2_b2_c2_b, s2_b2_c3_w, s2_b2_c3_b, s2_b3_c1_w, s2_b3_c1_b, s2_b3_c2_w, s2_b3_c2_b, s2_b3_c3_w, s2_b3_c3_b, s2_b4_c1_w, s2_b4_c1_b, s2_b4_c2_w, s2_b4_c2_b, s2_b4_c3_w, s2_b4_c3_b, s2_b5_c1_w, s2_b5_c1_b, s2_b5_c2_w, s2_b5_c2_b, s2_b5_c3_w, s2_b5_c3_b, s3_b0_c1_w, s3_b0_c1_b, s3_b0_c2_w, s3_b0_c2_b, s3_b0_c3_w, s3_b0_c3_b, s3_b0_ds_w, s3_b0_ds_b, s3_b1_c1_w, s3_b1_c1_b, s3_b1_c2_w, s3_b1_c2_b, s3_b1_c3_w, s3_b1_c3_b, s3_b2_c1_w, s3_b2_c1_b, s3_b2_c2_w, s3_b2_c2_b, s3_b2_c3_w, s3_b2_c3_b, fc_w, fc_b):
    A = dict(locals())
    t = jnp.transpose(x, (0, 2, 3, 1)).astype(jnp.bfloat16)
    t = _stem_pool(t, stem_w, stem_b)

    n_blocks = (3, 4, 6, 3)
    strides = (1, 2, 2, 2)
    batch_group = ((1, 1), (2, 4), (4, 8), (8, 8))
    sizes = (56, 56, 28, 14)
    for si in range(4):
        for bi in range(n_blocks[si]):
            args = [A[f's{si}_b{bi}_{c}_{t2}'] for c in ('c1', 'c2', 'c3')
                    for t2 in ('w', 'b')]
            f_in = si >= 1
            f_out = f_in or (si == 0 and bi == n_blocks[0] - 1)
            if bi == 0:
                hw = (sizes[si], sizes[si]) if f_in else None
                t = _bottleneck(t, *args, A[f's{si}_b{bi}_ds_w'],
                                A[f's{si}_b{bi}_ds_b'],
                                stride=strides[si], nb=batch_group[si][0],
                                hw=hw, flat_out=f_out)
            else:
                s = sizes[si] // strides[si]
                hw = (s, s) if f_in else None
                t = _bottleneck(t, *args, stride=1, nb=batch_group[si][1],
                                hw=hw, flat_out=f_out)

    return _gap_fc(t.reshape(x.shape[0], 49, 2048), fc_w, fc_b, 200)
